# Initial kernel scaffold; baseline (speedup 1.0000x reference)
import jax
import jax.numpy as jnp
from jax.experimental import pallas as pl


def kernel(node_features, edge_features, eW1, eb1, eg1, ebt1, eW2, eb2, eg2, ebt2, nW1, nb1, ng1, nbt1, nW2, nb2, ng2, nbt2, edge_index):
    raise NotImplementedError("write your pallas kernel here")


# speedup vs baseline: 3.5792x; 3.5792x over previous
import functools

import jax
import jax.numpy as jnp
import numpy as np
from jax import lax
from jax.experimental import pallas as pl
from jax.experimental.pallas import tpu as pltpu
from jax.experimental.pallas import tpu_sc as plsc

N_NODES = 10000
N_EDGES = 320000
D_NODE = 128
D_EDGE = 16

_NC, _NS = 2, 16
_NW = _NC * _NS
_CH = 128
_NCHUNK = N_EDGES // _CH
_SLOTS = -(-_NCHUNK // _NW)
_ZCH = 200
_NZ = N_NODES // _ZCH
_ZSLOTS = -(-_NZ // _NS)

_MESH = dict(core_axis_name="c", subcore_axis_name="s")


def _sc_gather(node_features, row, col):
    @functools.partial(
        pl.kernel,
        out_type=(
            jax.ShapeDtypeStruct((N_EDGES, D_NODE), jnp.float32),
            jax.ShapeDtypeStruct((N_EDGES, D_NODE), jnp.float32),
        ),
        scratch_types=[
            pltpu.VMEM((_CH,), jnp.int32),
            pltpu.VMEM((_CH,), jnp.int32),
            pltpu.VMEM((_CH, D_NODE), jnp.float32),
            pltpu.VMEM((_CH, D_NODE), jnp.float32),
            pltpu.SemaphoreType.DMA,
        ],
        mesh=plsc.VectorSubcoreMesh(**_MESH),
    )
    def gather_k(nf, row_h, col_h, src_o, tgt_o, idx_r, idx_c, buf_s, buf_t, sem):
        wid = lax.axis_index("s") * _NC + lax.axis_index("c")

        def body(g, c):
            chunk = wid + _NW * g

            @pl.when(chunk < _NCHUNK)
            def _():
                base = chunk * _CH
                pltpu.sync_copy(row_h.at[pl.ds(base, _CH)], idx_r)
                pltpu.sync_copy(col_h.at[pl.ds(base, _CH)], idx_c)
                a = pltpu.async_copy(nf.at[idx_r], buf_s, sem)
                b = pltpu.async_copy(nf.at[idx_c], buf_t, sem)
                a.wait()
                b.wait()
                pltpu.sync_copy(buf_s, src_o.at[pl.ds(base, _CH)])
                pltpu.sync_copy(buf_t, tgt_o.at[pl.ds(base, _CH)])

            return c

        lax.fori_loop(0, _SLOTS, body, 0)

    return gather_k(node_features, row, col)


def _sc_scatter(e_out, col):
    @functools.partial(
        pl.kernel,
        out_type=jax.ShapeDtypeStruct((_NC, N_NODES, D_NODE), jnp.float32),
        scratch_types=[
            pltpu.VMEM((_CH,), jnp.int32),
            pltpu.VMEM((_CH, D_NODE), jnp.float32),
            pltpu.VMEM((_ZCH, D_NODE), jnp.float32),
            pltpu.VMEM_SHARED((N_NODES, D_NODE), jnp.float32),
        ],
        mesh=plsc.VectorSubcoreMesh(**_MESH),
    )
    def scatter_k(eout_h, col_h, agg_o, idx_c, buf, zbuf, agg_sh):
        cid = lax.axis_index("c")
        sid = lax.axis_index("s")
        wid = sid * _NC + cid

        def zrow(i, c):
            def zcol(j, c2):
                zbuf[i, pl.ds(j * 16, 16)] = jnp.zeros((16,), jnp.float32)
                return c2

            return lax.fori_loop(0, D_NODE // 16, zcol, c)

        lax.fori_loop(0, _ZCH, zrow, 0)

        def zcp(k, c):
            zc = sid + _NS * k

            @pl.when(zc < _NZ)
            def _():
                pltpu.sync_copy(zbuf, agg_sh.at[pl.ds(zc * _ZCH, _ZCH)])

            return c

        lax.fori_loop(0, _ZSLOTS, zcp, 0)
        plsc.subcore_barrier()

        def body(g, c):
            chunk = wid + _NW * g

            @pl.when(chunk < _NCHUNK)
            def _():
                base = chunk * _CH
                pltpu.sync_copy(col_h.at[pl.ds(base, _CH)], idx_c)
                pltpu.sync_copy(eout_h.at[pl.ds(base, _CH)], buf)
                pltpu.sync_copy(buf, agg_sh.at[idx_c], add=True)

            return c

        lax.fori_loop(0, _SLOTS, body, 0)
        plsc.subcore_barrier()

        def cp(k, c):
            zc = sid + _NS * k

            @pl.when(zc < _NZ)
            def _():
                off = zc * _ZCH
                pltpu.sync_copy(agg_sh.at[pl.ds(off, _ZCH)], zbuf)
                pltpu.sync_copy(zbuf, agg_o.at[cid, pl.ds(off, _ZCH)])

            return c

        lax.fori_loop(0, _ZSLOTS, cp, 0)

    return scatter_k(e_out, col)


def _ln_gelu(x, g, b):
    m = jnp.mean(x, axis=-1, keepdims=True)
    c = x - m
    v = jnp.mean(c * c, axis=-1, keepdims=True)
    y = c * lax.rsqrt(v + 1e-5) * g + b
    return 0.5 * y * (1.0 + lax.erf(y * (1.0 / np.sqrt(2.0))))


_BE = 512


def _edge_body(src, tgt, ef, wsd, wtd, we, wdc, eb1, eg1, ebt1,
               ew2, eb2, eg2, ebt2, out):
    s = src[...]
    t = tgt[...]
    st = jnp.sum(s * t, axis=1, keepdims=True)
    ss = jnp.sum(s * s, axis=1, keepdims=True)
    tt = jnp.sum(t * t, axis=1, keepdims=True)
    d = s - t
    dist = jnp.sqrt(jnp.sum(d * d, axis=1, keepdims=True) + 1e-12)
    cos = st / (jnp.sqrt(ss + 1e-12) * jnp.sqrt(tt + 1e-12))
    h = jnp.dot(s, wsd[...], preferred_element_type=jnp.float32)
    h = h + jnp.dot(t, wtd[...], preferred_element_type=jnp.float32)
    h = h + jnp.dot(ef[...], we[...], preferred_element_type=jnp.float32)
    h = h + dist * wdc[0:1, :] + cos * wdc[1:2, :] + eb1[...]
    h = _ln_gelu(h, eg1[...], ebt1[...])
    h2 = jnp.dot(h, ew2[...], preferred_element_type=jnp.float32) + eb2[...]
    out[...] = _ln_gelu(h2, eg2[...], ebt2[...])


def _edge_mlp(src, tgt, ef, wsd, wtd, we, wdc, eb1, eg1, ebt1,
              ew2, eb2, eg2, ebt2):
    n_blk = N_EDGES // _BE
    row_spec = pl.BlockSpec((_BE, D_NODE), lambda i: (i, 0))
    const = lambda shape: pl.BlockSpec(shape, lambda i: (0, 0))
    return pl.pallas_call(
        _edge_body,
        grid=(n_blk,),
        in_specs=[
            row_spec,
            row_spec,
            pl.BlockSpec((_BE, D_EDGE), lambda i: (i, 0)),
            const((D_NODE, D_NODE)),
            const((D_NODE, D_NODE)),
            const((D_EDGE, D_NODE)),
            const((2, D_NODE)),
            const((1, D_NODE)),
            const((1, D_NODE)),
            const((1, D_NODE)),
            const((D_NODE, D_NODE)),
            const((1, D_NODE)),
            const((1, D_NODE)),
            const((1, D_NODE)),
        ],
        out_specs=row_spec,
        out_shape=jax.ShapeDtypeStruct((N_EDGES, D_NODE), jnp.float32),
    )(src, tgt, ef, wsd, wtd, we, wdc, eb1, eg1, ebt1, ew2, eb2, eg2, ebt2)


_BN = 1000


def _node_body(nf, agg, nwx, nwa, nb1, ng1, nbt1, nw2, nb2, ng2, nbt2, out):
    x = nf[...]
    a = agg[0] + agg[1]
    h = jnp.dot(x, nwx[...], preferred_element_type=jnp.float32)
    h = h + jnp.dot(a, nwa[...], preferred_element_type=jnp.float32) + nb1[...]
    h = _ln_gelu(h, ng1[...], nbt1[...])
    h2 = jnp.dot(h, nw2[...], preferred_element_type=jnp.float32) + nb2[...]
    out[...] = _ln_gelu(h2, ng2[...], nbt2[...])


def _node_mlp(nf, agg2, nwx, nwa, nb1, ng1, nbt1, nw2, nb2, ng2, nbt2):
    n_blk = N_NODES // _BN
    row_spec = pl.BlockSpec((_BN, D_NODE), lambda i: (i, 0))
    const = lambda shape: pl.BlockSpec(shape, lambda i: (0, 0))
    return pl.pallas_call(
        _node_body,
        grid=(n_blk,),
        in_specs=[
            row_spec,
            pl.BlockSpec((_NC, _BN, D_NODE), lambda i: (0, i, 0)),
            const((D_NODE, D_NODE)),
            const((D_NODE, D_NODE)),
            const((1, D_NODE)),
            const((1, D_NODE)),
            const((1, D_NODE)),
            const((D_NODE, D_NODE)),
            const((1, D_NODE)),
            const((1, D_NODE)),
            const((1, D_NODE)),
        ],
        out_specs=row_spec,
        out_shape=jax.ShapeDtypeStruct((N_NODES, D_NODE), jnp.float32),
    )(nf, agg2, nwx, nwa, nb1, ng1, nbt1, nw2, nb2, ng2, nbt2)


def kernel(node_features, edge_features,
           eW1, eb1, eg1, ebt1, eW2, eb2, eg2, ebt2,
           nW1, nb1, ng1, nbt1, nW2, nb2, ng2, nbt2,
           edge_index):
    ei = edge_index.astype(jnp.int32)
    row = ei[0]
    col = ei[1]

    wsd = eW1[0:D_NODE] + eW1[2 * D_NODE:3 * D_NODE]
    wtd = eW1[D_NODE:2 * D_NODE] - eW1[2 * D_NODE:3 * D_NODE]
    wdc = eW1[3 * D_NODE:3 * D_NODE + 2]
    we = eW1[3 * D_NODE + 2:]

    src, tgt = _sc_gather(node_features, row, col)
    e_out = _edge_mlp(
        src, tgt, edge_features, wsd, wtd, we, wdc,
        eb1.reshape(1, -1), eg1.reshape(1, -1), ebt1.reshape(1, -1),
        eW2, eb2.reshape(1, -1), eg2.reshape(1, -1), ebt2.reshape(1, -1))
    agg2 = _sc_scatter(e_out, col)
    x_out = _node_mlp(
        node_features, agg2, nW1[0:D_NODE], nW1[D_NODE:],
        nb1.reshape(1, -1), ng1.reshape(1, -1), nbt1.reshape(1, -1),
        nW2, nb2.reshape(1, -1), ng2.reshape(1, -1), nbt2.reshape(1, -1))
    return (x_out, e_out)
